# Initial kernel scaffold; baseline (speedup 1.0000x reference)
import jax
import jax.numpy as jnp
from jax.experimental import pallas as pl


def kernel(source_UV, source_VU, target_UV, target_VU, source_UU_adj, source_VV_adj, target_UU_adj, target_VV_adj, source_user_table, source_item_table, target_user_table, target_item_table, s_W_user, s_W_item, s_W_out_u, s_W_out_i, t_W_user, t_W_item, t_W_out_u, t_W_out_i):
    raise NotImplementedError("write your pallas kernel here")


# speedup vs baseline: 3.9365x; 3.9365x over previous
import functools

import jax
import jax.numpy as jnp
from jax import lax
from jax.experimental import pallas as pl
from jax.experimental.pallas import tpu as pltpu
from jax.experimental.pallas import tpu_sc as plsc

U = 10000
D = 128
E = 320000
NC = 2
NS = 16
NW = NC * NS
EPT = E // NW
CHUNK = 80
NCHUNK = EPT // CHUNK
ROWS_PT = 632
UPAD = NS * ROWS_PT
ZROWS = 8


def _spmm4(tables, dsts, srcs):
    mesh = plsc.VectorSubcoreMesh(core_axis_name="c", subcore_axis_name="s")
    out_type = [jax.ShapeDtypeStruct((NC, UPAD, D), jnp.float32)
                for _ in range(4)]
    scratch = [
        pltpu.VMEM((CHUNK,), jnp.int32),
        pltpu.VMEM((CHUNK,), jnp.int32),
        pltpu.VMEM((CHUNK, D), jnp.float32),
        pltpu.VMEM((ZROWS, D), jnp.float32),
        pltpu.VMEM_SHARED((UPAD, D), jnp.float32),
        pltpu.SemaphoreType.DMA,
    ]

    @functools.partial(pl.kernel, mesh=mesh, out_type=out_type,
                       scratch_types=scratch)
    def k(t0, t1, t2, t3, d0, d1, d2, d3, s0, s1, s2, s3,
          o0, o1, o2, o3, dst_v, src_v, rows_v, zbuf, acc, sem):
        cid = lax.axis_index("c")
        sid = lax.axis_index("s")
        wid = sid * NC + cid
        ebase = wid * EPT
        rbase = sid * ROWS_PT

        def zrow(i, _):
            def zcol(j, _):
                zbuf[i, pl.ds(j * 16, 16)] = jnp.zeros((16,), jnp.float32)
                return 0
            return lax.fori_loop(0, D // 16, zcol, 0)
        lax.fori_loop(0, ZROWS, zrow, 0)

        for tbl, dh, sh, out in ((t0, d0, s0, o0), (t1, d1, s1, o1),
                                 (t2, d2, s2, o2), (t3, d3, s3, o3)):
            def zcopy(z, _):
                pltpu.sync_copy(zbuf, acc.at[pl.ds(rbase + z * ZROWS, ZROWS)])
                return 0
            lax.fori_loop(0, ROWS_PT // ZROWS, zcopy, 0)
            plsc.subcore_barrier()

            def chunk(c, _):
                off = ebase + c * CHUNK
                pltpu.sync_copy(dh.at[pl.ds(off, CHUNK)], dst_v)
                pltpu.sync_copy(sh.at[pl.ds(off, CHUNK)], src_v)
                pltpu.async_copy(tbl.at[src_v], rows_v, sem).wait()
                pltpu.sync_copy(rows_v, acc.at[dst_v], add=True)
                return 0
            lax.fori_loop(0, NCHUNK, chunk, 0)
            plsc.subcore_barrier()

            pltpu.sync_copy(acc.at[pl.ds(rbase, ROWS_PT)],
                            out.at[cid, pl.ds(rbase, ROWS_PT)])

    return k(*tables, *dsts, *srcs)


def _mm(p, w, relu):
    blk = 1000

    def body(p_ref, w_ref, o_ref):
        x = p_ref[0] + p_ref[1]
        y = jnp.dot(x, w_ref[...], preferred_element_type=jnp.float32)
        if relu:
            y = jnp.maximum(y, 0.0)
        o_ref[...] = y

    return pl.pallas_call(
        body,
        grid=(U // blk,),
        in_specs=[pl.BlockSpec((NC, blk, D), lambda i: (0, i, 0)),
                  pl.BlockSpec((D, D), lambda i: (0, 0))],
        out_specs=pl.BlockSpec((blk, D), lambda i: (i, 0)),
        out_shape=jax.ShapeDtypeStruct((U, D), jnp.float32),
    )(p, w)


def kernel(source_UV, source_VU, target_UV, target_VU,
           source_UU_adj, source_VV_adj, target_UU_adj, target_VV_adj,
           source_user_table, source_item_table,
           target_user_table, target_item_table,
           s_W_user, s_W_item, s_W_out_u, s_W_out_i,
           t_W_user, t_W_item, t_W_out_u, t_W_out_i):
    def e(a):
        return jnp.asarray(a, jnp.int32)

    aggu_s, aggi_s, aggu_t, aggi_t = _spmm4(
        (source_item_table, source_user_table,
         target_item_table, target_user_table),
        (e(source_UV[0]), e(source_VU[0]), e(target_UV[0]), e(target_VU[0])),
        (e(source_UV[1]), e(source_VU[1]), e(target_UV[1]), e(target_VU[1])))

    hu_s = _mm(aggu_s, s_W_user, True)
    hi_s = _mm(aggi_s, s_W_item, True)
    hu_t = _mm(aggu_t, t_W_user, True)
    hi_t = _mm(aggi_t, t_W_item, True)

    hu2_s, hi2_s, hu2_t, hi2_t = _spmm4(
        (hu_s, hi_s, hu_t, hi_t),
        (e(source_UU_adj[0]), e(source_VV_adj[0]),
         e(target_UU_adj[0]), e(target_VV_adj[0])),
        (e(source_UU_adj[1]), e(source_VV_adj[1]),
         e(target_UU_adj[1]), e(target_VV_adj[1])))

    su = _mm(hu2_s, s_W_out_u, False)
    si = _mm(hi2_s, s_W_out_i, False)
    tu = _mm(hu2_t, t_W_out_u, False)
    ti = _mm(hi2_t, t_W_out_i, False)
    return (su, si, tu, ti)


# speedup vs baseline: 9.1359x; 2.3208x over previous
import functools

import jax
import jax.numpy as jnp
from jax import lax
from jax.experimental import pallas as pl
from jax.experimental.pallas import tpu as pltpu
from jax.experimental.pallas import tpu_sc as plsc

U = 10000
D = 128
E = 320000
NC = 2
NS = 16
NW = NC * NS
EPT = E // NW
CHUNK = 80
NCHUNK = EPT // CHUNK
ROWS_PT = 632
UPAD = NS * ROWS_PT
ZROWS = 8


def _spmm4(tables, dsts, srcs):
    mesh = plsc.VectorSubcoreMesh(core_axis_name="c", subcore_axis_name="s")
    out_type = [jax.ShapeDtypeStruct((NC, UPAD, D), jnp.float32)
                for _ in range(4)]
    scratch = [
        pltpu.VMEM((NCHUNK, CHUNK), jnp.int32),
        pltpu.VMEM((EPT,), jnp.int32),
        pltpu.VMEM((CHUNK, D), jnp.float32),
        pltpu.VMEM((CHUNK, D), jnp.float32),
        pltpu.VMEM((ZROWS, D), jnp.float32),
        pltpu.VMEM_SHARED((UPAD, D), jnp.float32),
        pltpu.SemaphoreType.DMA,
        pltpu.SemaphoreType.DMA,
    ]

    @functools.partial(pl.kernel, mesh=mesh, out_type=out_type,
                       scratch_types=scratch)
    def k(t0, t1, t2, t3, d0, d1, d2, d3, s0, s1, s2, s3,
          o0, o1, o2, o3, dst_v, src_v, rows0, rows1, zbuf, acc,
          sem0, sem1):
        cid = lax.axis_index("c")
        sid = lax.axis_index("s")
        wid = sid * NC + cid
        rbase = sid * ROWS_PT

        def zrow(i, _):
            def zcol(j, _):
                zbuf[i, pl.ds(j * 16, 16)] = jnp.zeros((16,), jnp.float32)
                return 0
            return lax.fori_loop(0, D // 16, zcol, 0)
        lax.fori_loop(0, ZROWS, zrow, 0)

        for tbl, dh, sh, out in ((t0, d0, s0, o0), (t1, d1, s1, o1),
                                 (t2, d2, s2, o2), (t3, d3, s3, o3)):
            def zcopy(z, _):
                pltpu.sync_copy(zbuf, acc.at[pl.ds(rbase + z * ZROWS, ZROWS)])
                return 0
            lax.fori_loop(0, ROWS_PT // ZROWS, zcopy, 0)
            pltpu.sync_copy(dh.at[wid], dst_v)
            pltpu.sync_copy(sh.at[wid], src_v)
            plsc.subcore_barrier()

            def sidx(c):
                return src_v.at[pl.ds(c * CHUNK, CHUNK)]

            pltpu.async_copy(tbl.at[sidx(0)], rows0, sem0)

            def step(i, _):
                c0 = 2 * i
                c1 = c0 + 1
                pltpu.async_copy(tbl.at[sidx(c1)], rows1, sem1)
                pltpu.make_async_copy(tbl.at[sidx(c0)], rows0, sem0).wait()
                pltpu.sync_copy(rows0, acc.at[dst_v.at[c0]], add=True)
                pltpu.async_copy(tbl.at[sidx(c1 + 1)], rows0, sem0)
                pltpu.make_async_copy(tbl.at[sidx(c1)], rows1, sem1).wait()
                pltpu.sync_copy(rows1, acc.at[dst_v.at[c1]], add=True)
                return 0
            lax.fori_loop(0, (NCHUNK - 1) // 2, step, 0)
            last = NCHUNK - 1
            pltpu.make_async_copy(tbl.at[sidx(last)], rows0, sem0).wait()
            pltpu.sync_copy(rows0, acc.at[dst_v.at[last]], add=True)
            plsc.subcore_barrier()

            pltpu.sync_copy(acc.at[pl.ds(rbase, ROWS_PT)],
                            out.at[cid, pl.ds(rbase, ROWS_PT)])

    return k(*tables, *dsts, *srcs)


def _mm(p, w, relu):
    blk = 1000

    def body(p_ref, w_ref, o_ref):
        x = p_ref[0] + p_ref[1]
        y = jnp.dot(x, w_ref[...], preferred_element_type=jnp.float32)
        if relu:
            y = jnp.maximum(y, 0.0)
        o_ref[...] = y

    return pl.pallas_call(
        body,
        grid=(U // blk,),
        in_specs=[pl.BlockSpec((NC, blk, D), lambda i: (0, i, 0)),
                  pl.BlockSpec((D, D), lambda i: (0, 0))],
        out_specs=pl.BlockSpec((blk, D), lambda i: (i, 0)),
        out_shape=jax.ShapeDtypeStruct((U, D), jnp.float32),
    )(p, w)


def kernel(source_UV, source_VU, target_UV, target_VU,
           source_UU_adj, source_VV_adj, target_UU_adj, target_VV_adj,
           source_user_table, source_item_table,
           target_user_table, target_item_table,
           s_W_user, s_W_item, s_W_out_u, s_W_out_i,
           t_W_user, t_W_item, t_W_out_u, t_W_out_i):
    def e(a):
        return jnp.asarray(a, jnp.int32).reshape(NW, NCHUNK, CHUNK)

    def f(a):
        return jnp.asarray(a, jnp.int32).reshape(NW, EPT)

    aggu_s, aggi_s, aggu_t, aggi_t = _spmm4(
        (source_item_table, source_user_table,
         target_item_table, target_user_table),
        (e(source_UV[0]), e(source_VU[0]), e(target_UV[0]), e(target_VU[0])),
        (f(source_UV[1]), f(source_VU[1]), f(target_UV[1]), f(target_VU[1])))

    hu_s = _mm(aggu_s, s_W_user, True)
    hi_s = _mm(aggi_s, s_W_item, True)
    hu_t = _mm(aggu_t, t_W_user, True)
    hi_t = _mm(aggi_t, t_W_item, True)

    hu2_s, hi2_s, hu2_t, hi2_t = _spmm4(
        (hu_s, hi_s, hu_t, hi_t),
        (e(source_UU_adj[0]), e(source_VV_adj[0]),
         e(target_UU_adj[0]), e(target_VV_adj[0])),
        (f(source_UU_adj[1]), f(source_VV_adj[1]),
         f(target_UU_adj[1]), f(target_VV_adj[1])))

    su = _mm(hu2_s, s_W_out_u, False)
    si = _mm(hi2_s, s_W_out_i, False)
    tu = _mm(hu2_t, t_W_out_u, False)
    ti = _mm(hi2_t, t_W_out_i, False)
    return (su, si, tu, ti)


# speedup vs baseline: 9.3486x; 1.0233x over previous
import functools

import jax
import jax.numpy as jnp
from jax import lax
from jax.experimental import pallas as pl
from jax.experimental.pallas import tpu as pltpu
from jax.experimental.pallas import tpu_sc as plsc

U = 10000
D = 128
E = 320000
NC = 2
NS = 16
NW = NC * NS
EPT = E // NW
CHUNK = 80
NCHUNK = EPT // CHUNK
ROWS_PT = 632
UPAD = NS * ROWS_PT


def _spmm4(tables, dsts, srcs, zrows):
    mesh = plsc.VectorSubcoreMesh(core_axis_name="c", subcore_axis_name="s")
    out_type = [jax.ShapeDtypeStruct((NC, UPAD, D), jnp.float32)
                for _ in range(4)]
    scratch = [
        pltpu.VMEM((NCHUNK, CHUNK), jnp.int32),
        pltpu.VMEM((EPT,), jnp.int32),
        pltpu.VMEM((CHUNK, D), jnp.float32),
        pltpu.VMEM((CHUNK, D), jnp.float32),
        pltpu.VMEM_SHARED((UPAD, D), jnp.float32),
        pltpu.SemaphoreType.DMA,
        pltpu.SemaphoreType.DMA,
        pltpu.SemaphoreType.DMA,
    ]

    @functools.partial(pl.kernel, mesh=mesh, out_type=out_type,
                       scratch_types=scratch)
    def k(zr, t0, t1, t2, t3, d0, d1, d2, d3, s0, s1, s2, s3,
          o0, o1, o2, o3, dst_v, src_v, rows0, rows1, acc,
          sem0, sem1, sem2):
        cid = lax.axis_index("c")
        sid = lax.axis_index("s")
        wid = sid * NC + cid
        rbase = sid * ROWS_PT

        for tbl, dh, sh, out in ((t0, d0, s0, o0), (t1, d1, s1, o1),
                                 (t2, d2, s2, o2), (t3, d3, s3, o3)):
            pltpu.async_copy(zr, acc.at[pl.ds(rbase, ROWS_PT)], sem2)
            pltpu.async_copy(dh.at[wid], dst_v, sem2)
            pltpu.async_copy(sh.at[wid], src_v, sem2)
            pltpu.make_async_copy(zr, acc.at[pl.ds(rbase, ROWS_PT)],
                                  sem2).wait()
            pltpu.make_async_copy(dh.at[wid], dst_v, sem2).wait()
            pltpu.make_async_copy(sh.at[wid], src_v, sem2).wait()
            plsc.subcore_barrier()

            def sidx(c):
                return src_v.at[pl.ds(c * CHUNK, CHUNK)]

            pltpu.async_copy(tbl.at[sidx(0)], rows0, sem0)

            def step(i, _):
                c0 = 2 * i
                c1 = c0 + 1
                pltpu.async_copy(tbl.at[sidx(c1)], rows1, sem1)
                pltpu.make_async_copy(tbl.at[sidx(c0)], rows0, sem0).wait()
                pltpu.sync_copy(rows0, acc.at[dst_v.at[c0]], add=True)
                pltpu.async_copy(tbl.at[sidx(c1 + 1)], rows0, sem0)
                pltpu.make_async_copy(tbl.at[sidx(c1)], rows1, sem1).wait()
                pltpu.sync_copy(rows1, acc.at[dst_v.at[c1]], add=True)
                return 0
            npairs = (NCHUNK - 1) // 2
            lax.fori_loop(0, npairs, step, 0)
            if NCHUNK % 2 == 0:
                c0 = NCHUNK - 2
                pltpu.async_copy(tbl.at[sidx(c0 + 1)], rows1, sem1)
                pltpu.make_async_copy(tbl.at[sidx(c0)], rows0, sem0).wait()
                pltpu.sync_copy(rows0, acc.at[dst_v.at[c0]], add=True)
                pltpu.make_async_copy(tbl.at[sidx(c0 + 1)], rows1,
                                      sem1).wait()
                pltpu.sync_copy(rows1, acc.at[dst_v.at[c0 + 1]], add=True)
            else:
                last = NCHUNK - 1
                pltpu.make_async_copy(tbl.at[sidx(last)], rows0, sem0).wait()
                pltpu.sync_copy(rows0, acc.at[dst_v.at[last]], add=True)
            plsc.subcore_barrier()

            pltpu.sync_copy(acc.at[pl.ds(rbase, ROWS_PT)],
                            out.at[cid, pl.ds(rbase, ROWS_PT)])

    return k(zrows, *tables, *dsts, *srcs)


def _mm(p, w, relu):
    blk = 1000

    def body(p_ref, w_ref, o_ref):
        x = p_ref[0] + p_ref[1]
        y = jnp.dot(x, w_ref[...], preferred_element_type=jnp.float32)
        if relu:
            y = jnp.maximum(y, 0.0)
        o_ref[...] = y

    return pl.pallas_call(
        body,
        grid=(U // blk,),
        in_specs=[pl.BlockSpec((NC, blk, D), lambda i: (0, i, 0)),
                  pl.BlockSpec((D, D), lambda i: (0, 0))],
        out_specs=pl.BlockSpec((blk, D), lambda i: (i, 0)),
        out_shape=jax.ShapeDtypeStruct((U, D), jnp.float32),
    )(p, w)


def kernel(source_UV, source_VU, target_UV, target_VU,
           source_UU_adj, source_VV_adj, target_UU_adj, target_VV_adj,
           source_user_table, source_item_table,
           target_user_table, target_item_table,
           s_W_user, s_W_item, s_W_out_u, s_W_out_i,
           t_W_user, t_W_item, t_W_out_u, t_W_out_i):
    def e(a):
        return jnp.asarray(a, jnp.int32).reshape(NW, NCHUNK, CHUNK)

    def f(a):
        return jnp.asarray(a, jnp.int32).reshape(NW, EPT)

    zrows = jnp.zeros((ROWS_PT, D), jnp.float32)

    aggu_s, aggi_s, aggu_t, aggi_t = _spmm4(
        (source_item_table, source_user_table,
         target_item_table, target_user_table),
        (e(source_UV[0]), e(source_VU[0]), e(target_UV[0]), e(target_VU[0])),
        (f(source_UV[1]), f(source_VU[1]), f(target_UV[1]), f(target_VU[1])),
        zrows)

    hu_s = _mm(aggu_s, s_W_user, True)
    hi_s = _mm(aggi_s, s_W_item, True)
    hu_t = _mm(aggu_t, t_W_user, True)
    hi_t = _mm(aggi_t, t_W_item, True)

    hu2_s, hi2_s, hu2_t, hi2_t = _spmm4(
        (hu_s, hi_s, hu_t, hi_t),
        (e(source_UU_adj[0]), e(source_VV_adj[0]),
         e(target_UU_adj[0]), e(target_VV_adj[0])),
        (f(source_UU_adj[1]), f(source_VV_adj[1]),
         f(target_UU_adj[1]), f(target_VV_adj[1])),
        zrows)

    su = _mm(hu2_s, s_W_out_u, False)
    si = _mm(hi2_s, s_W_out_i, False)
    tu = _mm(hu2_t, t_W_out_u, False)
    ti = _mm(hi2_t, t_W_out_i, False)
    return (su, si, tu, ti)
